# baseline (speedup 1.0000x reference)
import functools

import jax
import jax.numpy as jnp
import numpy as np
from jax import lax
from jax.experimental import pallas as pl
from jax.experimental.pallas import tpu as pltpu
from jax.experimental.pallas import tpu_sc as plsc

_N = 16
_W = 384
_H = 384
_NUM_SAMPLES = 2000
_TOTAL = _N * _NUM_SAMPLES
_NW = 32
_PER_W = 1024
_PAD = _NW * _PER_W
_LANES = 16
_CHUNKS = _PER_W // _LANES
_THRESH2 = float(np.float32(0.1) * np.float32(0.1))


def _prep_constants():
    key = jax.random.key(42)
    k1, k2, k3 = jax.random.split(key, 3)
    us, vs = [], []
    for k in (k1, k2, k3):
        ku, kv = jax.random.split(k)
        us.append(jax.random.randint(ku, (_N, _NUM_SAMPLES), 0, _W))
        vs.append(jax.random.randint(kv, (_N, _NUM_SAMPLES), 0, _H))
    boff = (jnp.arange(_N, dtype=jnp.int32) * (_W * _H))[:, None]
    idx = [(boff + u * _H + v).reshape(-1) for u, v in zip(us, vs)]
    uf = [(u.astype(jnp.float32) / _W).reshape(-1) for u in us]
    vf = [(v.astype(jnp.float32) / _H).reshape(-1) for v in vs]
    dx12 = uf[1] - uf[0]
    dy12 = vf[1] - vf[0]
    dx13 = uf[2] - uf[0]
    dy13 = vf[2] - vf[0]
    npad = _PAD - _TOTAL

    def pad(a):
        return jnp.concatenate([a, jnp.zeros((npad,), a.dtype)])

    return tuple(pad(a) for a in (idx[0], idx[1], idx[2], dx12, dy12, dx13, dy13))


def _rsqrt(s):
    i = lax.bitcast_convert_type(s, jnp.int32)
    i = jnp.int32(0x5F3759DF) - (i >> 1)
    y = lax.bitcast_convert_type(i, jnp.float32)
    for _ in range(3):
        y = y * (1.5 - 0.5 * s * y * y)
    return y


@functools.cache
def _make_sc_kernel():
    mesh = plsc.VectorSubcoreMesh(core_axis_name="c", subcore_axis_name="s")
    f32, i32 = jnp.float32, jnp.int32

    @functools.partial(
        pl.kernel,
        mesh=mesh,
        out_type=[
            jax.ShapeDtypeStruct((_NW, _LANES), f32),
            jax.ShapeDtypeStruct((_NW, _LANES), f32),
        ],
        scratch_types=[
            pltpu.VMEM((_PER_W,), i32),
            pltpu.VMEM((_PER_W,), i32),
            pltpu.VMEM((_PER_W,), i32),
            pltpu.VMEM((_PER_W,), f32),
            pltpu.VMEM((_PER_W,), f32),
            pltpu.VMEM((_PER_W,), f32),
            pltpu.VMEM((_PER_W,), f32),
            pltpu.VMEM((_PER_W,), f32),
            pltpu.VMEM((_PER_W,), f32),
            pltpu.VMEM((_PER_W,), f32),
            pltpu.VMEM((_PER_W,), f32),
            pltpu.VMEM((_PER_W,), f32),
            pltpu.VMEM((_PER_W,), f32),
            pltpu.VMEM((_LANES,), f32),
            pltpu.VMEM((_LANES,), f32),
            pltpu.SemaphoreType.DMA,
            pltpu.SemaphoreType.DMA,
            pltpu.SemaphoreType.DMA,
        ],
    )
    def sc_kernel(pred_hbm, targ_hbm, i1h, i2h, i3h, axh, ayh, bxh, byh,
                  out_sum, out_cnt,
                  i1v, i2v, i3v, axv, ayv, bxv, byv,
                  d1p, d2p, d3p, d1t, d2t, d3t,
                  acc_s, acc_c, sem_idx, sem_geo, sem_gat):
        wid = lax.axis_index("s") * 2 + lax.axis_index("c")
        base = wid * _PER_W
        sl = pl.ds(base, _PER_W)
        idx_cp = [
            pltpu.async_copy(i1h.at[sl], i1v, sem_idx),
            pltpu.async_copy(i2h.at[sl], i2v, sem_idx),
            pltpu.async_copy(i3h.at[sl], i3v, sem_idx),
        ]
        geo_cp = [
            pltpu.async_copy(axh.at[sl], axv, sem_geo),
            pltpu.async_copy(ayh.at[sl], ayv, sem_geo),
            pltpu.async_copy(bxh.at[sl], bxv, sem_geo),
            pltpu.async_copy(byh.at[sl], byv, sem_geo),
        ]
        for c in idx_cp:
            c.wait()
        gat_cp = [
            pltpu.async_copy(pred_hbm.at[i1v], d1p, sem_gat),
            pltpu.async_copy(pred_hbm.at[i2v], d2p, sem_gat),
            pltpu.async_copy(pred_hbm.at[i3v], d3p, sem_gat),
            pltpu.async_copy(targ_hbm.at[i1v], d1t, sem_gat),
            pltpu.async_copy(targ_hbm.at[i2v], d2t, sem_gat),
            pltpu.async_copy(targ_hbm.at[i3v], d3t, sem_gat),
        ]
        for c in geo_cp:
            c.wait()
        for c in gat_cp:
            c.wait()

        def body(i, carry):
            s_acc, c_acc = carry
            ch = pl.ds(i * _LANES, _LANES)
            ax = axv[ch]
            ay = ayv[ch]
            bx = bxv[ch]
            by = byv[ch]
            t1 = d1t[ch]
            e12t = d2t[ch] - t1
            e13t = d3t[ch] - t1
            p1 = d1p[ch]
            e12p = d2p[ch] - p1
            e13p = d3p[ch] - p1
            cz = ax * by - ay * bx
            cxt = ay * e13t - e12t * by
            cyt = e12t * bx - ax * e13t
            cxp = ay * e13p - e12p * by
            cyp = e12p * bx - ax * e13p
            st = cxt * cxt + cyt * cyt + cz * cz
            sp = cxp * cxp + cyp * cyp + cz * cz
            mask = ((st > _THRESH2) & (t1 > 0.0)
                    & (d2t[ch] > 0.0) & (d3t[ch] > 0.0))
            ft = _rsqrt(jnp.maximum(st, 1e-24))
            fp = _rsqrt(jnp.maximum(sp, 1e-24))
            contrib = (jnp.abs(cxp * fp - cxt * ft)
                       + jnp.abs(cyp * fp - cyt * ft)
                       + jnp.abs(cz * fp - cz * ft))
            s_acc = s_acc + jnp.where(mask, contrib, 0.0)
            c_acc = c_acc + jnp.where(mask, 1.0, 0.0)
            return s_acc, c_acc

        zero = jnp.zeros((_LANES,), f32)
        s_acc, c_acc = lax.fori_loop(0, _CHUNKS, body, (zero, zero))
        acc_s[...] = s_acc
        acc_c[...] = c_acc
        pltpu.sync_copy(acc_s, out_sum.at[wid])
        pltpu.sync_copy(acc_c, out_cnt.at[wid])

    return sc_kernel


def _reduce_body(s_ref, c_ref, o_ref):
    total = jnp.sum(s_ref[...])
    valid = jnp.sum(c_ref[...])
    res = total / jnp.maximum(valid * 3.0, 1.0)
    o_ref[...] = jnp.reshape(res, (1, 1))


def kernel(pred, target):
    pred_f = pred.reshape(-1)
    targ_f = target.reshape(-1)
    i1, i2, i3, ax, ay, bx, by = _prep_constants()
    sums, cnts = _make_sc_kernel()(pred_f, targ_f, i1, i2, i3, ax, ay, bx, by)
    out = pl.pallas_call(
        _reduce_body,
        out_shape=jax.ShapeDtypeStruct((1, 1), jnp.float32),
    )(sums, cnts)
    return out[0, 0]


# speedup vs baseline: 2.2238x; 2.2238x over previous
import functools

import jax
import jax.numpy as jnp
import numpy as np
from jax import lax
from jax.experimental import pallas as pl
from jax.experimental.pallas import tpu as pltpu
from jax.experimental.pallas import tpu_sc as plsc

_N = 16
_W = 384
_H = 384
_NUM_SAMPLES = 2000
_TOTAL = _N * _NUM_SAMPLES
_NW = 32
_PER_W = 1024
_PAD = _NW * _PER_W
_LANES = 16
_CHUNKS = _PER_W // _LANES
_THRESH2 = float(np.float32(0.1) * np.float32(0.1))


def _prep_constants():
    key = jax.random.key(42)
    k1, k2, k3 = jax.random.split(key, 3)
    us, vs = [], []
    for k in (k1, k2, k3):
        ku, kv = jax.random.split(k)
        us.append(jax.random.randint(ku, (_N, _NUM_SAMPLES), 0, _W))
        vs.append(jax.random.randint(kv, (_N, _NUM_SAMPLES), 0, _H))
    boff = (jnp.arange(_N, dtype=jnp.int32) * (_W * _H))[:, None]
    idx = [(boff + u * _H + v).reshape(-1) for u, v in zip(us, vs)]
    uf = [(u.astype(jnp.float32) / _W).reshape(-1) for u in us]
    vf = [(v.astype(jnp.float32) / _H).reshape(-1) for v in vs]
    dx12 = uf[1] - uf[0]
    dy12 = vf[1] - vf[0]
    dx13 = uf[2] - uf[0]
    dy13 = vf[2] - vf[0]
    npad = _PAD - _TOTAL

    def pad(a):
        return jnp.concatenate([a, jnp.zeros((npad,), a.dtype)])

    return tuple(pad(a) for a in (idx[0], idx[1], idx[2], dx12, dy12, dx13, dy13))


@functools.cache
def _host_constants():
    try:
        dev = jax.devices("cpu")[0]
    except RuntimeError:
        dev = None
    with jax.ensure_compile_time_eval():
        if dev is not None:
            with jax.default_device(dev):
                vals = _prep_constants()
        else:
            vals = _prep_constants()
    return tuple(np.asarray(v) for v in jax.device_get(vals))


def _rsqrt(s):
    i = lax.bitcast_convert_type(s, jnp.int32)
    i = jnp.int32(0x5F3759DF) - (i >> 1)
    y = lax.bitcast_convert_type(i, jnp.float32)
    for _ in range(3):
        y = y * (1.5 - 0.5 * s * y * y)
    return y


@functools.cache
def _make_sc_kernel():
    mesh = plsc.VectorSubcoreMesh(core_axis_name="c", subcore_axis_name="s")
    f32, i32 = jnp.float32, jnp.int32

    @functools.partial(
        pl.kernel,
        mesh=mesh,
        out_type=[
            jax.ShapeDtypeStruct((_NW, _LANES), f32),
            jax.ShapeDtypeStruct((_NW, _LANES), f32),
        ],
        scratch_types=[
            pltpu.VMEM((_PER_W,), i32),
            pltpu.VMEM((_PER_W,), i32),
            pltpu.VMEM((_PER_W,), i32),
            pltpu.VMEM((_PER_W,), f32),
            pltpu.VMEM((_PER_W,), f32),
            pltpu.VMEM((_PER_W,), f32),
            pltpu.VMEM((_PER_W,), f32),
            pltpu.VMEM((_PER_W,), f32),
            pltpu.VMEM((_PER_W,), f32),
            pltpu.VMEM((_PER_W,), f32),
            pltpu.VMEM((_PER_W,), f32),
            pltpu.VMEM((_PER_W,), f32),
            pltpu.VMEM((_PER_W,), f32),
            pltpu.VMEM((_LANES,), f32),
            pltpu.VMEM((_LANES,), f32),
            pltpu.SemaphoreType.DMA,
            pltpu.SemaphoreType.DMA,
            pltpu.SemaphoreType.DMA,
        ],
    )
    def sc_kernel(pred_hbm, targ_hbm, i1h, i2h, i3h, axh, ayh, bxh, byh,
                  out_sum, out_cnt,
                  i1v, i2v, i3v, axv, ayv, bxv, byv,
                  d1p, d2p, d3p, d1t, d2t, d3t,
                  acc_s, acc_c, sem_idx, sem_geo, sem_gat):
        wid = lax.axis_index("s") * 2 + lax.axis_index("c")
        base = wid * _PER_W
        sl = pl.ds(base, _PER_W)
        idx_cp = [
            pltpu.async_copy(i1h.at[sl], i1v, sem_idx),
            pltpu.async_copy(i2h.at[sl], i2v, sem_idx),
            pltpu.async_copy(i3h.at[sl], i3v, sem_idx),
        ]
        geo_cp = [
            pltpu.async_copy(axh.at[sl], axv, sem_geo),
            pltpu.async_copy(ayh.at[sl], ayv, sem_geo),
            pltpu.async_copy(bxh.at[sl], bxv, sem_geo),
            pltpu.async_copy(byh.at[sl], byv, sem_geo),
        ]
        for c in idx_cp:
            c.wait()
        gat_cp = [
            pltpu.async_copy(pred_hbm.at[i1v], d1p, sem_gat),
            pltpu.async_copy(pred_hbm.at[i2v], d2p, sem_gat),
            pltpu.async_copy(pred_hbm.at[i3v], d3p, sem_gat),
            pltpu.async_copy(targ_hbm.at[i1v], d1t, sem_gat),
            pltpu.async_copy(targ_hbm.at[i2v], d2t, sem_gat),
            pltpu.async_copy(targ_hbm.at[i3v], d3t, sem_gat),
        ]
        for c in geo_cp:
            c.wait()
        for c in gat_cp:
            c.wait()

        def body(i, carry):
            s_acc, c_acc = carry
            ch = pl.ds(i * _LANES, _LANES)
            ax = axv[ch]
            ay = ayv[ch]
            bx = bxv[ch]
            by = byv[ch]
            t1 = d1t[ch]
            e12t = d2t[ch] - t1
            e13t = d3t[ch] - t1
            p1 = d1p[ch]
            e12p = d2p[ch] - p1
            e13p = d3p[ch] - p1
            cz = ax * by - ay * bx
            cxt = ay * e13t - e12t * by
            cyt = e12t * bx - ax * e13t
            cxp = ay * e13p - e12p * by
            cyp = e12p * bx - ax * e13p
            st = cxt * cxt + cyt * cyt + cz * cz
            sp = cxp * cxp + cyp * cyp + cz * cz
            mask = ((st > _THRESH2) & (t1 > 0.0)
                    & (d2t[ch] > 0.0) & (d3t[ch] > 0.0))
            ft = _rsqrt(jnp.maximum(st, 1e-24))
            fp = _rsqrt(jnp.maximum(sp, 1e-24))
            contrib = (jnp.abs(cxp * fp - cxt * ft)
                       + jnp.abs(cyp * fp - cyt * ft)
                       + jnp.abs(cz * fp - cz * ft))
            s_acc = s_acc + jnp.where(mask, contrib, 0.0)
            c_acc = c_acc + jnp.where(mask, 1.0, 0.0)
            return s_acc, c_acc

        zero = jnp.zeros((_LANES,), f32)
        s_acc, c_acc = lax.fori_loop(0, _CHUNKS, body, (zero, zero))
        acc_s[...] = s_acc
        acc_c[...] = c_acc
        pltpu.sync_copy(acc_s, out_sum.at[wid])
        pltpu.sync_copy(acc_c, out_cnt.at[wid])

    return sc_kernel


def _reduce_body(s_ref, c_ref, o_ref):
    total = jnp.sum(s_ref[...])
    valid = jnp.sum(c_ref[...])
    res = total / jnp.maximum(valid * 3.0, 1.0)
    o_ref[...] = jnp.reshape(res, (1, 1))


def kernel(pred, target):
    pred_f = pred.reshape(-1)
    targ_f = target.reshape(-1)
    i1, i2, i3, ax, ay, bx, by = (jnp.asarray(c) for c in _host_constants())
    sums, cnts = _make_sc_kernel()(pred_f, targ_f, i1, i2, i3, ax, ay, bx, by)
    out = pl.pallas_call(
        _reduce_body,
        out_shape=jax.ShapeDtypeStruct((1, 1), jnp.float32),
    )(sums, cnts)
    return out[0, 0]


# speedup vs baseline: 2.3248x; 1.0454x over previous
import functools

import jax
import jax.numpy as jnp
import numpy as np
from jax import lax
from jax.experimental import pallas as pl
from jax.experimental.pallas import tpu as pltpu
from jax.experimental.pallas import tpu_sc as plsc

_N = 16
_W = 384
_H = 384
_NUM_SAMPLES = 2000
_TOTAL = _N * _NUM_SAMPLES
_NW = 32
_PER_W = 1024
_PAD = _NW * _PER_W
_HALF = _PER_W // 2
_LANES = 16
_HCHUNKS = _HALF // _LANES
_THRESH2 = float(np.float32(0.1) * np.float32(0.1))


def _prep_constants():
    key = jax.random.key(42)
    k1, k2, k3 = jax.random.split(key, 3)
    us, vs = [], []
    for k in (k1, k2, k3):
        ku, kv = jax.random.split(k)
        us.append(jax.random.randint(ku, (_N, _NUM_SAMPLES), 0, _W))
        vs.append(jax.random.randint(kv, (_N, _NUM_SAMPLES), 0, _H))
    boff = (jnp.arange(_N, dtype=jnp.int32) * (_W * _H))[:, None]
    idx = [(boff + u * _H + v).reshape(-1) for u, v in zip(us, vs)]
    uf = [(u.astype(jnp.float32) / _W).reshape(-1) for u in us]
    vf = [(v.astype(jnp.float32) / _H).reshape(-1) for v in vs]
    dx12 = uf[1] - uf[0]
    dy12 = vf[1] - vf[0]
    dx13 = uf[2] - uf[0]
    dy13 = vf[2] - vf[0]
    npad = _PAD - _TOTAL

    def pad(a):
        return jnp.concatenate([a, jnp.zeros((npad,), a.dtype)])

    return tuple(pad(a) for a in (idx[0], idx[1], idx[2], dx12, dy12, dx13, dy13))


def _pack_constants(vals):
    i1, i2, i3, ax, ay, bx, by = vals
    icat = jnp.stack([v.reshape(2 * _NW, _HALF) for v in (i1, i2, i3)],
                     axis=1).reshape(2 * _NW, 3 * _HALF)
    geo = jnp.stack([v.reshape(_NW, _PER_W) for v in (ax, ay, bx, by)],
                    axis=1).reshape(_NW, 4 * _PER_W)
    return icat, geo


@functools.cache
def _host_constants():
    try:
        try:
            dev = jax.devices("cpu")[0]
        except RuntimeError:
            dev = None
        with jax.ensure_compile_time_eval():
            if dev is not None:
                with jax.default_device(dev):
                    vals = _pack_constants(_prep_constants())
            else:
                vals = _pack_constants(_prep_constants())
        return tuple(np.asarray(v) for v in jax.device_get(vals))
    except Exception:
        return None


def _rsqrt(s):
    i = lax.bitcast_convert_type(s, jnp.int32)
    i = jnp.int32(0x5F3759DF) - (i >> 1)
    y = lax.bitcast_convert_type(i, jnp.float32)
    for _ in range(3):
        y = y * (1.5 - 0.5 * s * y * y)
    return y


@functools.cache
def _make_sc_kernel():
    mesh = plsc.VectorSubcoreMesh(core_axis_name="c", subcore_axis_name="s")
    f32, i32 = jnp.float32, jnp.int32

    @functools.partial(
        pl.kernel,
        mesh=mesh,
        out_type=jax.ShapeDtypeStruct((_NW, 2 * _LANES), f32),
        scratch_types=[
            pltpu.VMEM((3 * _HALF,), i32),
            pltpu.VMEM((3 * _HALF,), i32),
            pltpu.VMEM((4 * _PER_W,), f32),
            pltpu.VMEM((3 * _HALF,), f32),
            pltpu.VMEM((3 * _HALF,), f32),
            pltpu.VMEM((3 * _HALF,), f32),
            pltpu.VMEM((3 * _HALF,), f32),
            pltpu.VMEM((2 * _LANES,), f32),
            pltpu.SemaphoreType.DMA,
            pltpu.SemaphoreType.DMA,
            pltpu.SemaphoreType.DMA,
            pltpu.SemaphoreType.DMA,
            pltpu.SemaphoreType.DMA,
        ],
    )
    def sc_kernel(pred_hbm, targ_hbm, icat_h, geo_h, out_h,
                  iA, iB, geo, gpA, gtA, gpB, gtB, acc,
                  sem_iA, sem_iB, sem_geo, sem_gA, sem_gB):
        wid = lax.axis_index("s") * 2 + lax.axis_index("c")
        cp_iA = pltpu.async_copy(icat_h.at[2 * wid], iA, sem_iA)
        cp_iB = pltpu.async_copy(icat_h.at[2 * wid + 1], iB, sem_iB)
        cp_geo = pltpu.async_copy(geo_h.at[wid], geo, sem_geo)
        cp_iA.wait()
        gA = [pltpu.async_copy(pred_hbm.at[iA], gpA, sem_gA),
              pltpu.async_copy(targ_hbm.at[iA], gtA, sem_gA)]
        cp_iB.wait()
        gB = [pltpu.async_copy(pred_hbm.at[iB], gpB, sem_gB),
              pltpu.async_copy(targ_hbm.at[iB], gtB, sem_gB)]
        cp_geo.wait()

        def make_body(gp, gt, half):
            def body(i, carry):
                s_acc, c_acc = carry
                off = i * _LANES
                smp = half * _HALF + off
                d1p = gp[pl.ds(off, _LANES)]
                d2p = gp[pl.ds(_HALF + off, _LANES)]
                d3p = gp[pl.ds(2 * _HALF + off, _LANES)]
                t1 = gt[pl.ds(off, _LANES)]
                t2 = gt[pl.ds(_HALF + off, _LANES)]
                t3 = gt[pl.ds(2 * _HALF + off, _LANES)]
                ax = geo[pl.ds(smp, _LANES)]
                ay = geo[pl.ds(_PER_W + smp, _LANES)]
                bx = geo[pl.ds(2 * _PER_W + smp, _LANES)]
                by = geo[pl.ds(3 * _PER_W + smp, _LANES)]
                e12t = t2 - t1
                e13t = t3 - t1
                e12p = d2p - d1p
                e13p = d3p - d1p
                cz = ax * by - ay * bx
                cxt = ay * e13t - e12t * by
                cyt = e12t * bx - ax * e13t
                cxp = ay * e13p - e12p * by
                cyp = e12p * bx - ax * e13p
                st = cxt * cxt + cyt * cyt + cz * cz
                sp = cxp * cxp + cyp * cyp + cz * cz
                mask = ((st > _THRESH2) & (t1 > 0.0) & (t2 > 0.0) & (t3 > 0.0))
                ft = _rsqrt(jnp.maximum(st, 1e-24))
                fp = _rsqrt(jnp.maximum(sp, 1e-24))
                contrib = (jnp.abs(cxp * fp - cxt * ft)
                           + jnp.abs(cyp * fp - cyt * ft)
                           + jnp.abs(cz * fp - cz * ft))
                s_acc = s_acc + jnp.where(mask, contrib, 0.0)
                c_acc = c_acc + jnp.where(mask, 1.0, 0.0)
                return s_acc, c_acc
            return body

        zero = jnp.zeros((_LANES,), f32)
        for c in gA:
            c.wait()
        accs = lax.fori_loop(0, _HCHUNKS, make_body(gpA, gtA, 0), (zero, zero))
        for c in gB:
            c.wait()
        accs = lax.fori_loop(0, _HCHUNKS, make_body(gpB, gtB, 1), accs)
        acc[pl.ds(0, _LANES)] = accs[0]
        acc[pl.ds(_LANES, _LANES)] = accs[1]
        pltpu.sync_copy(acc, out_h.at[wid])

    return sc_kernel


def _reduce_body(p_ref, o_ref):
    total = jnp.sum(p_ref[:, 0:_LANES])
    valid = jnp.sum(p_ref[:, _LANES:2 * _LANES])
    res = total / jnp.maximum(valid * 3.0, 1.0)
    o_ref[...] = jnp.reshape(res, (1, 1))


def kernel(pred, target):
    pred_f = pred.reshape(-1)
    targ_f = target.reshape(-1)
    consts = _host_constants()
    if consts is None:
        icat, geo = _pack_constants(_prep_constants())
    else:
        icat, geo = (jnp.asarray(c) for c in consts)
    parts = _make_sc_kernel()(pred_f, targ_f, icat, geo)
    out = pl.pallas_call(
        _reduce_body,
        out_shape=jax.ShapeDtypeStruct((1, 1), jnp.float32),
    )(parts)
    return out[0, 0]
